# baseline (speedup 1.0000x reference)
import functools

import jax
import jax.numpy as jnp
from jax import lax
from jax.experimental import pallas as pl
from jax.experimental.pallas import tpu as pltpu
from jax.experimental.pallas import tpu_sc as plsc

T = 4
B = 4096
L = 200
D = 16
N = B * L

NC = 2
NS = 16
NW = NC * NS
PER_W = N // NW
CH = 2560
NCHUNK = PER_W // CH

_mesh = plsc.VectorSubcoreMesh(
    core_axis_name="c", subcore_axis_name="s", num_cores=NC, num_subcores=NS
)


@functools.partial(
    pl.kernel,
    out_type=jax.ShapeDtypeStruct((T * N, D), jnp.float32),
    mesh=_mesh,
    scratch_types=[
        pltpu.VMEM((CH,), jnp.int32),
        pltpu.VMEM((CH, D), jnp.float32),
        pltpu.SemaphoreType.DMA,
    ],
    compiler_params=pltpu.CompilerParams(use_tc_tiling_on_sc=False),
)
def _sc_embed(w_hbm, idx_hbm, out_hbm, idx_v, rows_v, sem):
    wid = lax.axis_index("s") * NC + lax.axis_index("c")
    base = wid * PER_W

    def chunk_body(c, carry):
        off = base + c * CH
        pltpu.sync_copy(idx_hbm.at[pl.ds(off, CH)], idx_v)
        pltpu.async_copy(w_hbm.at[idx_v], rows_v, sem).wait()

        def scale_body(i, carry2):
            rows_v[i, :] = rows_v[i, :] * jnp.float32(1.0 / T)
            return carry2

        lax.fori_loop(0, CH, scale_body, 0, unroll=8)
        for t in range(T):
            pltpu.sync_copy(rows_v, out_hbm.at[pl.ds(t * N + off, CH)])
        return carry

    lax.fori_loop(0, NCHUNK, chunk_body, 0)


def kernel(input, weight):
    idx_flat = input.reshape(N)
    out_flat = _sc_embed(weight, idx_flat)
    return out_flat.reshape(T * B, L, D)


# speedup vs baseline: 1.0093x; 1.0093x over previous
import functools

import jax
import jax.numpy as jnp
from jax import lax
from jax.experimental import pallas as pl
from jax.experimental.pallas import tpu as pltpu
from jax.experimental.pallas import tpu_sc as plsc

T = 4
B = 4096
L = 200
D = 16
N = B * L

NC = 2
NS = 16
NW = NC * NS
PER_W = N // NW
CH = 2560
NCHUNK = PER_W // CH
NBUF = 2
SCALE = 1.0 / T

_mesh = plsc.VectorSubcoreMesh(
    core_axis_name="c", subcore_axis_name="s", num_cores=NC, num_subcores=NS
)


@functools.partial(
    pl.kernel,
    out_type=jax.ShapeDtypeStruct((T * N, D), jnp.float32),
    mesh=_mesh,
    scratch_types=[
        pltpu.VMEM((NBUF, CH), jnp.int32),
        pltpu.VMEM((NBUF, CH, D), jnp.float32),
        pltpu.SemaphoreType.DMA,
        pltpu.SemaphoreType.DMA,
        pltpu.SemaphoreType.DMA,
        pltpu.SemaphoreType.DMA,
    ],
    compiler_params=pltpu.CompilerParams(use_tc_tiling_on_sc=False),
)
def _sc_embed(w_hbm, idx_hbm, out_hbm, idx_v, rows_v, g0, g1, s0, s1):
    wid = lax.axis_index("s") * NC + lax.axis_index("c")
    base = wid * PER_W
    gsem = (g0, g1)
    ssem = (s0, s1)

    def fetch(c, buf):
        off = base + c * CH
        pltpu.sync_copy(idx_hbm.at[pl.ds(off, CH)], idx_v.at[buf])
        return pltpu.async_copy(w_hbm.at[idx_v.at[buf]], rows_v.at[buf], gsem[buf])

    gather_descs = [None, None]
    store_descs = [[], []]
    gather_descs[0] = fetch(0, 0)
    for c in range(NCHUNK):
        buf = c & 1
        nbuf = 1 - buf
        if c + 1 < NCHUNK:
            for d in store_descs[nbuf]:
                d.wait()
            store_descs[nbuf] = []
            gather_descs[nbuf] = fetch(c + 1, nbuf)
        gather_descs[buf].wait()

        def scale_body(i, carry, _buf=buf):
            rows_v[_buf, i, :] = rows_v[_buf, i, :] * jnp.float32(SCALE)
            return carry

        lax.fori_loop(0, CH, scale_body, 0, unroll=8)

        off = base + c * CH
        store_descs[buf] = [
            pltpu.async_copy(
                rows_v.at[buf], out_hbm.at[pl.ds(t * N + off, CH)], ssem[buf]
            )
            for t in range(T)
        ]
    for buf in range(NBUF):
        for d in store_descs[buf]:
            d.wait()


def kernel(input, weight):
    idx_flat = input.reshape(N)
    out_flat = _sc_embed(weight, idx_flat)
    return out_flat.reshape(T * B, L, D)


# speedup vs baseline: 2.3645x; 2.3427x over previous
import functools

import jax
import jax.numpy as jnp
from jax import lax
from jax.experimental import pallas as pl
from jax.experimental.pallas import tpu as pltpu
from jax.experimental.pallas import tpu_sc as plsc

T = 4
B = 4096
L = 200
D = 16
N = B * L

NC = 2
NS = 16
NW = NC * NS
CH = 1024
NCHUNK_TOTAL = N // CH
PER_W = NCHUNK_TOTAL // NW
NBUF = 2
SCALE = 1.0 / T

_mesh = plsc.VectorSubcoreMesh(
    core_axis_name="c", subcore_axis_name="s", num_cores=NC, num_subcores=NS
)


@functools.partial(
    pl.kernel,
    out_type=jax.ShapeDtypeStruct((L, D, T * B), jnp.float32),
    mesh=_mesh,
    scratch_types=[
        pltpu.VMEM((NBUF, CH), jnp.int32),
        pltpu.VMEM((NBUF, CH, D), jnp.float32),
        pltpu.VMEM((NBUF, D, CH), jnp.float32),
        pltpu.SemaphoreType.DMA,
        pltpu.SemaphoreType.DMA,
        pltpu.SemaphoreType.DMA,
        pltpu.SemaphoreType.DMA,
    ],
    compiler_params=pltpu.CompilerParams(
        use_tc_tiling_on_sc=False, needs_layout_passes=False
    ),
)
def _sc_embed(w_hbm, idx_hbm, out_hbm, idx_v, rows_v, tr_v, g0, g1, s0, s1):
    wid = lax.axis_index("s") * NC + lax.axis_index("c")
    gsem = (g0, g1)
    ssem = (s0, s1)
    lane_iota = lax.iota(jnp.int32, 16)

    def fetch(k, buf):
        cg = wid * PER_W + k
        pltpu.sync_copy(idx_hbm.at[pl.ds(cg * CH, CH)], idx_v.at[buf])
        return pltpu.async_copy(w_hbm.at[idx_v.at[buf]], rows_v.at[buf], gsem[buf])

    gather_descs = [None, None]
    store_descs = [[], []]
    gather_descs[0] = fetch(0, 0)
    for k in range(PER_W):
        buf = k & 1
        if k + 1 < PER_W:
            gather_descs[1 - buf] = fetch(k + 1, 1 - buf)
        for d in store_descs[buf]:
            d.wait()
        gather_descs[buf].wait()

        def tr_body(p, carry, _buf=buf):
            row = rows_v[_buf, p, :] * jnp.float32(SCALE)
            pcol = jnp.broadcast_to(p, (16,)).astype(jnp.int32)
            plsc.store_scatter(tr_v.at[_buf], [lane_iota, pcol], row)
            return carry

        lax.fori_loop(0, CH, tr_body, 0, unroll=8)

        cg = wid * PER_W + k
        l = cg // (B // CH)
        b0 = (cg % (B // CH)) * CH
        store_descs[buf] = [
            pltpu.async_copy(
                tr_v.at[buf],
                out_hbm.at[l, :, pl.ds(t * B + b0, CH)],
                ssem[buf],
            )
            for t in range(T)
        ]
    for buf in range(NBUF):
        for d in store_descs[buf]:
            d.wait()


def kernel(input, weight):
    idx_flat = input.T.reshape(N)
    out_ldb = _sc_embed(weight, idx_flat)
    return jnp.transpose(out_ldb, (2, 0, 1))


# speedup vs baseline: 2.8276x; 1.1958x over previous
import functools

import jax
import jax.numpy as jnp
from jax import lax
from jax.experimental import pallas as pl
from jax.experimental.pallas import tpu as pltpu
from jax.experimental.pallas import tpu_sc as plsc

T = 4
B = 4096
L = 200
D = 16
N = B * L

NC = 2
NS = 16
NW = NC * NS
CH = 1024
NCHUNK_TOTAL = N // CH
PER_W = NCHUNK_TOTAL // NW
NBUF = 2
SCALE = 1.0 / T

_mesh = plsc.VectorSubcoreMesh(
    core_axis_name="c", subcore_axis_name="s", num_cores=NC, num_subcores=NS
)

VOCAB = 1000000
_TC_BC = 16384


def _tc_prep_body(wt_ref, out_ref):
    x = wt_ref[...] * jnp.float32(SCALE)
    xt = jnp.transpose(x)
    x3 = xt.reshape(_TC_BC // 8, 8, D)
    parts = [x3[:, k, :] for k in range(8)]
    out_ref[...] = jnp.concatenate(parts, axis=1)


def _tc_prep(wt):
    nb = pl.cdiv(VOCAB, _TC_BC)
    return pl.pallas_call(
        _tc_prep_body,
        grid=(nb,),
        in_specs=[pl.BlockSpec((D, _TC_BC), lambda i: (0, i))],
        out_specs=pl.BlockSpec((_TC_BC // 8, 128), lambda i: (i, 0)),
        out_shape=jax.ShapeDtypeStruct((VOCAB // 8, 128), jnp.float32),
    )(wt)


@functools.partial(
    pl.kernel,
    out_type=jax.ShapeDtypeStruct((L, D, T * B), jnp.float32),
    mesh=_mesh,
    scratch_types=[
        pltpu.VMEM((NBUF, CH), jnp.int32),
        pltpu.VMEM((NBUF, CH, D), jnp.float32),
        pltpu.VMEM((NBUF, D, CH), jnp.float32),
        pltpu.SemaphoreType.DMA,
        pltpu.SemaphoreType.DMA,
        pltpu.SemaphoreType.DMA,
        pltpu.SemaphoreType.DMA,
    ],
    compiler_params=pltpu.CompilerParams(
        use_tc_tiling_on_sc=False, needs_layout_passes=False
    ),
)
def _sc_embed(w_hbm, idx_hbm, out_hbm, idx_v, rows_v, tr_v, g0, g1, s0, s1):
    wid = lax.axis_index("s") * NC + lax.axis_index("c")
    gsem = (g0, g1)
    ssem = (s0, s1)
    lane_iota = lax.iota(jnp.int32, 16)

    def fetch(k, buf):
        cg = wid * PER_W + k
        pltpu.sync_copy(idx_hbm.at[pl.ds(cg * CH, CH)], idx_v.at[buf])
        return pltpu.async_copy(w_hbm.at[idx_v.at[buf]], rows_v.at[buf], gsem[buf])

    gather_descs = [None, None]
    store_descs = [[], []]
    gather_descs[0] = fetch(0, 0)
    for k in range(PER_W):
        buf = k & 1
        if k + 1 < PER_W:
            gather_descs[1 - buf] = fetch(k + 1, 1 - buf)
        for d in store_descs[buf]:
            d.wait()
        gather_descs[buf].wait()

        def tr_body(p, carry, _buf=buf):
            row = rows_v[_buf, p, :]
            pcol = jnp.broadcast_to(p, (16,)).astype(jnp.int32)
            plsc.store_scatter(tr_v.at[_buf], [lane_iota, pcol], row)
            return carry

        lax.fori_loop(0, CH, tr_body, 0, unroll=8)

        cg = wid * PER_W + k
        l = cg // (B // CH)
        b0 = (cg % (B // CH)) * CH
        store_descs[buf] = [
            pltpu.async_copy(
                tr_v.at[buf],
                out_hbm.at[l, :, pl.ds(t * B + b0, CH)],
                ssem[buf],
            )
            for t in range(T)
        ]
    for buf in range(NBUF):
        for d in store_descs[buf]:
            d.wait()


def kernel(input, weight):
    w_lin = _tc_prep(jnp.transpose(weight)).reshape(VOCAB, D)
    idx_flat = input.T.reshape(N)
    out_ldb = _sc_embed(w_lin, idx_flat)
    return jnp.transpose(out_ldb, (2, 0, 1))


# speedup vs baseline: 3.8419x; 1.3587x over previous
import functools

import jax
import jax.numpy as jnp
from jax import lax
from jax.experimental import pallas as pl
from jax.experimental.pallas import tpu as pltpu
from jax.experimental.pallas import tpu_sc as plsc

T = 4
B = 4096
L = 200
D = 16
N = B * L

NC = 2
NS = 16
NW = NC * NS
CH = 1024
NCHUNK_TOTAL = N // CH
PER_W = NCHUNK_TOTAL // NW
NBUF = 2
SCALE = 1.0 / T

_mesh = plsc.VectorSubcoreMesh(
    core_axis_name="c", subcore_axis_name="s", num_cores=NC, num_subcores=NS
)

VOCAB = 1000000
_TC_BC = 16384


def _tc_prep_body(wt_ref, out_ref):
    x = wt_ref[...] * jnp.float32(SCALE)
    xt = jnp.transpose(x)
    x3 = xt.reshape(_TC_BC // 8, 8, D)
    parts = [x3[:, k, :] for k in range(8)]
    out_ref[...] = jnp.concatenate(parts, axis=1)


def _tc_prep(wt):
    nb = pl.cdiv(VOCAB, _TC_BC)
    return pl.pallas_call(
        _tc_prep_body,
        grid=(nb,),
        in_specs=[pl.BlockSpec((D, _TC_BC), lambda i: (0, i))],
        out_specs=pl.BlockSpec((_TC_BC // 8, 128), lambda i: (i, 0)),
        out_shape=jax.ShapeDtypeStruct((VOCAB // 8, 128), jnp.float32),
    )(wt)


NJ = T * B // 128
JCH = CH // 128


@functools.partial(
    pl.kernel,
    out_type=jax.ShapeDtypeStruct((L, D // 8, NJ, 8, 128), jnp.float32),
    mesh=_mesh,
    scratch_types=[
        pltpu.VMEM((NBUF, CH), jnp.int32),
        pltpu.VMEM((NBUF, CH, D), jnp.float32),
        pltpu.VMEM((NBUF, 2, JCH, 8, 128), jnp.float32),
        pltpu.SemaphoreType.DMA,
        pltpu.SemaphoreType.DMA,
        pltpu.SemaphoreType.DMA,
        pltpu.SemaphoreType.DMA,
    ],
    compiler_params=pltpu.CompilerParams(
        use_tc_tiling_on_sc=False, needs_layout_passes=False
    ),
)
def _sc_embed(w_hbm, idx_hbm, out_hbm, idx_v, rows_v, tr_v, g0, g1, s0, s1):
    wid = lax.axis_index("s") * NC + lax.axis_index("c")
    gsem = (g0, g1)
    ssem = (s0, s1)
    lane_iota = lax.iota(jnp.int32, 16)
    i_vec = lax.shift_right_logical(lane_iota, 3)
    s_vec = lax.bitwise_and(lane_iota, 7)

    def fetch(k, buf):
        cg = wid * PER_W + k
        pltpu.sync_copy(idx_hbm.at[pl.ds(cg * CH, CH)], idx_v.at[buf])
        return pltpu.async_copy(w_hbm.at[idx_v.at[buf]], rows_v.at[buf], gsem[buf])

    gather_descs = [None, None]
    store_descs = [[], []]
    gather_descs[0] = fetch(0, 0)
    for k in range(PER_W):
        buf = k & 1
        if k + 1 < PER_W:
            gather_descs[1 - buf] = fetch(k + 1, 1 - buf)
        for d in store_descs[buf]:
            d.wait()
        gather_descs[buf].wait()

        def tr_body(p, carry, _buf=buf):
            row = rows_v[_buf, p, :]
            j_b = jnp.broadcast_to(lax.shift_right_logical(p, 7), (16,))
            c_b = jnp.broadcast_to(lax.bitwise_and(p, 127), (16,))
            plsc.store_scatter(
                tr_v.at[_buf], [i_vec, j_b.astype(jnp.int32), s_vec,
                                c_b.astype(jnp.int32)], row
            )
            return carry

        lax.fori_loop(0, CH, tr_body, 0, unroll=8)

        cg = wid * PER_W + k
        l = cg // (B // CH)
        b0 = (cg % (B // CH)) * CH
        descs = []
        for t in range(T):
            j0 = (t * B + b0) // 128
            for i in range(2):
                descs.append(
                    pltpu.async_copy(
                        tr_v.at[buf, i],
                        out_hbm.at[l, i, pl.ds(j0, JCH)],
                        ssem[buf],
                    )
                )
        store_descs[buf] = descs
    for buf in range(NBUF):
        for d in store_descs[buf]:
            d.wait()


def kernel(input, weight):
    w_lin = _tc_prep(jnp.transpose(weight)).reshape(VOCAB, D)
    idx_flat = input.T.reshape(N)
    out5 = _sc_embed(w_lin, idx_flat)
    t1 = jnp.transpose(out5, (2, 4, 0, 1, 3))
    return t1.reshape(T * B, L, D)
